# baseline (speedup 1.0000x reference)
import functools

import numpy as np
import jax
import jax.numpy as jnp
from jax import lax
from jax.experimental import pallas as pl
from jax.experimental.pallas import tpu as pltpu
from jax.experimental.pallas import tpu_sc as plsc

N = 8192
INPUT_DIM = 4
HID = 256
OUT = 4
NF = 10
NL = 8
NC = 8
B = 256
NBLK = N // B + NC
NROWS = NBLK * B
PADW = 128
CHUNK = 80
NCHUNK = NROWS // CHUNK
NCC, NSC = 2, 16
NW = NCC * NSC
CPW = NCHUNK // NW

_PERM_SIN = np.array([4 + i * 8 + 2 * j for i in range(NF) for j in range(INPUT_DIM)])
_PERM_COS = _PERM_SIN + 1
_F2 = np.zeros((3 * INPUT_DIM, NF * INPUT_DIM), np.float32)
for _i in range(NF):
    for _j in range(INPUT_DIM):
        for _k in range(3):
            _F2[_k * INPUT_DIM + _j, _i * INPUT_DIM + _j] = 2.0 ** _i


def _mlp_block(be_ref, tok_ref, f2_ref, w0a_ref, w0sc_ref, b0_ref,
               wb_ref, bb_ref, sc_ref, wo_ref, bo_ref, out_ref):
    t = tok_ref[...]
    x4 = t[:, 0:4]
    xn = jnp.where(lax.broadcasted_iota(jnp.int32, (B, INPUT_DIM), 1) < 3,
                   x4 / t[:, 3:4], x4)
    x1 = xn.astype(jnp.bfloat16).astype(jnp.float32)
    r1 = xn - x1
    x2 = r1.astype(jnp.bfloat16).astype(jnp.float32)
    x3 = r1 - x2
    xs = jnp.concatenate([x1, x2, x3], axis=1)
    ang = jnp.dot(xs, f2_ref[...],
                  preferred_element_type=jnp.float32) * np.float32(np.pi)
    angc = jnp.concatenate([ang, ang + np.float32(np.pi / 2)], axis=1)
    s80 = jnp.sin(angc)
    h = (jnp.dot(xn, w0a_ref[0], preferred_element_type=jnp.float32)
         + jnp.dot(s80, w0sc_ref[0], preferred_element_type=jnp.float32)
         + b0_ref[0])
    h = jnp.maximum(h, 0.0)
    for l in range(NL - 1):
        z = jnp.dot(h, wb_ref[0, l], preferred_element_type=jnp.float32) + bb_ref[0, l]
        h = sc_ref[0, l] * jnp.maximum(z, 0.0) + h
    o = jnp.dot(h, wo_ref[0], preferred_element_type=jnp.float32) + bo_ref[0]
    out_ref[...] = o / t[:, 4:5]


@functools.cache
def _sc_kernels():
    mesh = plsc.VectorSubcoreMesh(core_axis_name="c", subcore_axis_name="s")

    @functools.partial(
        pl.kernel, mesh=mesh,
        out_type=jax.ShapeDtypeStruct((NROWS, PADW), jnp.float32),
        scratch_types=[pltpu.VMEM((CPW, CHUNK), jnp.int32),
                       pltpu.VMEM((CPW, CHUNK, PADW), jnp.float32),
                       pltpu.SemaphoreType.DMA,
                       pltpu.SemaphoreType.DMA,
                       pltpu.SemaphoreType.DMA])
    def sc_gather(table, idx2, out, idx_v, rows_v, s1, s2, s3):
        wid = lax.axis_index("s") * NCC + lax.axis_index("c")
        base = wid * CPW
        pltpu.async_copy(idx2.at[pl.ds(base, CPW)], idx_v, s1).wait()
        gcs = [pltpu.async_copy(table.at[idx_v.at[k]], rows_v.at[k], s2)
               for k in range(CPW)]
        wcs = []
        for k in range(CPW):
            gcs[k].wait()
            wcs.append(pltpu.async_copy(
                rows_v.at[k], out.at[pl.ds((base + k) * CHUNK, CHUNK)], s3))
        for w in wcs:
            w.wait()

    @functools.partial(
        pl.kernel, mesh=mesh,
        out_type=jax.ShapeDtypeStruct((N + 8, PADW), jnp.float32),
        scratch_types=[pltpu.VMEM((CPW, CHUNK), jnp.int32),
                       pltpu.VMEM((CPW, CHUNK, PADW), jnp.float32),
                       pltpu.SemaphoreType.DMA,
                       pltpu.SemaphoreType.DMA,
                       pltpu.SemaphoreType.DMA])
    def sc_scatter(vals, idx2, out, idx_v, rows_v, s1, s2, s3):
        wid = lax.axis_index("s") * NCC + lax.axis_index("c")
        base = wid * CPW
        ic = pltpu.async_copy(idx2.at[pl.ds(base, CPW)], idx_v, s1)
        vcs = [pltpu.async_copy(vals.at[pl.ds((base + k) * CHUNK, CHUNK)],
                                rows_v.at[k], s2)
               for k in range(CPW)]
        ic.wait()
        wcs = []
        for k in range(CPW):
            vcs[k].wait()
            wcs.append(pltpu.async_copy(rows_v.at[k], out.at[idx_v.at[k]], s3))
        for w in wcs:
            w.wait()

    return sc_gather, sc_scatter


def _routing(layer_id):
    e = layer_id.astype(jnp.int32)
    onehot = (e[:, None] == jnp.arange(NC, dtype=jnp.int32)).astype(jnp.int32)
    rank = jnp.take_along_axis(jnp.cumsum(onehot, axis=0) - onehot,
                               e[:, None], axis=1)[:, 0]
    counts = jnp.sum(onehot, axis=0)
    padded = ((counts + B - 1) // B) * B
    starts = jnp.concatenate(
        [jnp.zeros((1,), jnp.int32), jnp.cumsum(padded)[:-1].astype(jnp.int32)])
    dest = jnp.take(starts, e) + rank
    sidx = jnp.full((NROWS,), N, jnp.int32).at[dest].set(
        jnp.arange(N, dtype=jnp.int32))
    gidx = jnp.where(sidx == N, 0, sidx)
    block_expert = (jnp.searchsorted(
        starts, jnp.arange(NBLK, dtype=jnp.int32) * B, side="right") - 1
    ).astype(jnp.int32)
    return gidx, sidx, block_expert


def kernel(x, layer_id, input_dim, W0, b0, Wb, bb, scalars, Wo, bo):
    gidx, sidx, block_expert = _routing(layer_id)

    xpad = jnp.concatenate(
        [x, input_dim[:, None], jnp.zeros((N, PADW - INPUT_DIM - 1), jnp.float32)],
        axis=1)

    sc_gather, sc_scatter = _sc_kernels()
    tok = sc_gather(xpad, gidx.reshape(NCHUNK, CHUNK))

    f2 = jnp.asarray(_F2)
    w0a = W0[:, :INPUT_DIM, :]
    w0sc = W0[:, np.concatenate([_PERM_SIN, _PERM_COS]), :]
    b0r = b0[:, None, :]
    scl3 = jnp.broadcast_to(scalars[:, :, None], (NC, NL - 1, HID))
    wo16 = jnp.zeros((NC, HID, PADW), jnp.float32).at[:, :, :OUT].set(Wo)
    bo16 = jnp.zeros((NC, 1, PADW), jnp.float32).at[:, 0, :OUT].set(bo)

    grid_spec = pltpu.PrefetchScalarGridSpec(
        num_scalar_prefetch=1,
        grid=(NBLK,),
        in_specs=[
            pl.BlockSpec((B, PADW), lambda i, be: (i, 0)),
            pl.BlockSpec((3 * INPUT_DIM, NF * INPUT_DIM), lambda i, be: (0, 0)),
            pl.BlockSpec((1, INPUT_DIM, HID), lambda i, be: (be[i], 0, 0)),
            pl.BlockSpec((1, 2 * NF * INPUT_DIM, HID), lambda i, be: (be[i], 0, 0)),
            pl.BlockSpec((1, 1, HID), lambda i, be: (be[i], 0, 0)),
            pl.BlockSpec((1, NL - 1, HID, HID), lambda i, be: (be[i], 0, 0, 0)),
            pl.BlockSpec((1, NL - 1, HID), lambda i, be: (be[i], 0, 0)),
            pl.BlockSpec((1, NL - 1, HID), lambda i, be: (be[i], 0, 0)),
            pl.BlockSpec((1, HID, PADW), lambda i, be: (be[i], 0, 0)),
            pl.BlockSpec((1, 1, PADW), lambda i, be: (be[i], 0, 0)),
        ],
        out_specs=pl.BlockSpec((B, PADW), lambda i, be: (i, 0)),
    )
    vals = pl.pallas_call(
        _mlp_block,
        grid_spec=grid_spec,
        out_shape=jax.ShapeDtypeStruct((NROWS, PADW), jnp.float32),
    )(block_expert, tok, f2, w0a, w0sc, b0r, Wb, bb, scl3, wo16, bo16)

    scat = sc_scatter(vals, sidx.reshape(NCHUNK, CHUNK))
    return scat[:N, :OUT]


# speedup vs baseline: 1.0280x; 1.0280x over previous
import functools

import numpy as np
import jax
import jax.numpy as jnp
from jax import lax
from jax.experimental import pallas as pl
from jax.experimental.pallas import tpu as pltpu
from jax.experimental.pallas import tpu_sc as plsc

N = 8192
INPUT_DIM = 4
HID = 256
OUT = 4
NF = 10
NL = 8
NC = 8
B = 256
NBLK = N // B + NC
NROWS = NBLK * B
PADW = 128
CHUNK = 80
NCHUNK = NROWS // CHUNK
NCC, NSC = 2, 16
NW = NCC * NSC
CPW = NCHUNK // NW

_PERM_SIN = np.array([4 + i * 8 + 2 * j for i in range(NF) for j in range(INPUT_DIM)])
_PERM_COS = _PERM_SIN + 1
_F2 = np.zeros((3 * INPUT_DIM, NF * INPUT_DIM), np.float32)
for _i in range(NF):
    for _j in range(INPUT_DIM):
        for _k in range(3):
            _F2[_k * INPUT_DIM + _j, _i * INPUT_DIM + _j] = 2.0 ** _i


def _mlp_block(be_ref, nv_ref, tok_ref, f2_ref, w0a_ref, w0sc_ref, b0_ref,
               wb_ref, bb_ref, sc_ref, wo_ref, bo_ref, out_ref):
    @pl.when(pl.program_id(0) < nv_ref[0])
    def _valid_block():
        _mlp_block_body(tok_ref, f2_ref, w0a_ref, w0sc_ref, b0_ref,
                        wb_ref, bb_ref, sc_ref, wo_ref, bo_ref, out_ref)


def _mlp_block_body(tok_ref, f2_ref, w0a_ref, w0sc_ref, b0_ref,
                    wb_ref, bb_ref, sc_ref, wo_ref, bo_ref, out_ref):
    t = tok_ref[...]
    x4 = t[:, 0:4]
    xn = jnp.where(lax.broadcasted_iota(jnp.int32, (B, INPUT_DIM), 1) < 3,
                   x4 / t[:, 3:4], x4)
    x1 = xn.astype(jnp.bfloat16).astype(jnp.float32)
    r1 = xn - x1
    x2 = r1.astype(jnp.bfloat16).astype(jnp.float32)
    x3 = r1 - x2
    xs = jnp.concatenate([x1, x2, x3], axis=1)
    ang = jnp.dot(xs, f2_ref[...],
                  preferred_element_type=jnp.float32) * np.float32(np.pi)
    angc = jnp.concatenate([ang, ang + np.float32(np.pi / 2)], axis=1)
    s80 = jnp.sin(angc)
    h = (jnp.dot(xn, w0a_ref[0], preferred_element_type=jnp.float32)
         + jnp.dot(s80, w0sc_ref[0], preferred_element_type=jnp.float32)
         + b0_ref[0])
    h = jnp.maximum(h, 0.0)
    for l in range(NL - 1):
        z = jnp.dot(h, wb_ref[0, l], preferred_element_type=jnp.float32) + bb_ref[0, l]
        h = sc_ref[0, l] * jnp.maximum(z, 0.0) + h
    o = jnp.dot(h, wo_ref[0], preferred_element_type=jnp.float32) + bo_ref[0]
    out_ref[...] = o / t[:, 4:5]


@functools.cache
def _sc_kernels():
    mesh = plsc.VectorSubcoreMesh(core_axis_name="c", subcore_axis_name="s")

    @functools.partial(
        pl.kernel, mesh=mesh,
        out_type=jax.ShapeDtypeStruct((NROWS, PADW), jnp.float32),
        scratch_types=[pltpu.VMEM((CPW, CHUNK), jnp.int32),
                       pltpu.VMEM((CPW, CHUNK, PADW), jnp.float32),
                       pltpu.SemaphoreType.DMA,
                       pltpu.SemaphoreType.DMA,
                       pltpu.SemaphoreType.DMA])
    def sc_gather(table, idx2, out, idx_v, rows_v, s1, s2, s3):
        wid = lax.axis_index("s") * NCC + lax.axis_index("c")
        base = wid * CPW
        pltpu.async_copy(idx2.at[pl.ds(base, CPW)], idx_v, s1).wait()
        gcs = [pltpu.async_copy(table.at[idx_v.at[k]], rows_v.at[k], s2)
               for k in range(CPW)]
        wcs = []
        for k in range(CPW):
            gcs[k].wait()
            wcs.append(pltpu.async_copy(
                rows_v.at[k], out.at[pl.ds((base + k) * CHUNK, CHUNK)], s3))
        for w in wcs:
            w.wait()

    @functools.partial(
        pl.kernel, mesh=mesh,
        out_type=jax.ShapeDtypeStruct((N + 8, PADW), jnp.float32),
        scratch_types=[pltpu.VMEM((CPW, CHUNK), jnp.int32),
                       pltpu.VMEM((CPW, CHUNK, PADW), jnp.float32),
                       pltpu.SemaphoreType.DMA,
                       pltpu.SemaphoreType.DMA,
                       pltpu.SemaphoreType.DMA])
    def sc_scatter(vals, idx2, out, idx_v, rows_v, s1, s2, s3):
        wid = lax.axis_index("s") * NCC + lax.axis_index("c")
        base = wid * CPW
        ic = pltpu.async_copy(idx2.at[pl.ds(base, CPW)], idx_v, s1)
        vcs = [pltpu.async_copy(vals.at[pl.ds((base + k) * CHUNK, CHUNK)],
                                rows_v.at[k], s2)
               for k in range(CPW)]
        ic.wait()
        wcs = []
        for k in range(CPW):
            vcs[k].wait()
            wcs.append(pltpu.async_copy(rows_v.at[k], out.at[idx_v.at[k]], s3))
        for w in wcs:
            w.wait()

    return sc_gather, sc_scatter


def _routing(layer_id):
    e = layer_id.astype(jnp.int32)
    onehot = (e[:, None] == jnp.arange(NC, dtype=jnp.int32)).astype(jnp.int32)
    rank = jnp.take_along_axis(jnp.cumsum(onehot, axis=0) - onehot,
                               e[:, None], axis=1)[:, 0]
    counts = jnp.sum(onehot, axis=0)
    padded = ((counts + B - 1) // B) * B
    starts = jnp.concatenate(
        [jnp.zeros((1,), jnp.int32), jnp.cumsum(padded)[:-1].astype(jnp.int32)])
    dest = jnp.take(starts, e) + rank
    sidx = jnp.full((NROWS,), N, jnp.int32).at[dest].set(
        jnp.arange(N, dtype=jnp.int32))
    gidx = jnp.where(sidx == N, 0, sidx)
    block_expert = (jnp.searchsorted(
        starts, jnp.arange(NBLK, dtype=jnp.int32) * B, side="right") - 1
    ).astype(jnp.int32)
    nvb = (jnp.sum(padded) // B).astype(jnp.int32)[None]
    return gidx, sidx, block_expert, nvb


def kernel(x, layer_id, input_dim, W0, b0, Wb, bb, scalars, Wo, bo):
    gidx, sidx, block_expert, nvb = _routing(layer_id)

    xpad = jnp.concatenate(
        [x, input_dim[:, None], jnp.zeros((N, PADW - INPUT_DIM - 1), jnp.float32)],
        axis=1)

    sc_gather, sc_scatter = _sc_kernels()
    tok = sc_gather(xpad, gidx.reshape(NCHUNK, CHUNK))

    f2 = jnp.asarray(_F2)
    w0a = W0[:, :INPUT_DIM, :]
    w0sc = W0[:, np.concatenate([_PERM_SIN, _PERM_COS]), :]
    b0r = b0[:, None, :]
    scl3 = jnp.broadcast_to(scalars[:, :, None], (NC, NL - 1, HID))
    wo16 = jnp.zeros((NC, HID, PADW), jnp.float32).at[:, :, :OUT].set(Wo)
    bo16 = jnp.zeros((NC, 1, PADW), jnp.float32).at[:, 0, :OUT].set(bo)

    grid_spec = pltpu.PrefetchScalarGridSpec(
        num_scalar_prefetch=2,
        grid=(NBLK,),
        in_specs=[
            pl.BlockSpec((B, PADW), lambda i, be, nv: (i, 0)),
            pl.BlockSpec((3 * INPUT_DIM, NF * INPUT_DIM), lambda i, be, nv: (0, 0)),
            pl.BlockSpec((1, INPUT_DIM, HID), lambda i, be, nv: (be[i], 0, 0)),
            pl.BlockSpec((1, 2 * NF * INPUT_DIM, HID), lambda i, be, nv: (be[i], 0, 0)),
            pl.BlockSpec((1, 1, HID), lambda i, be, nv: (be[i], 0, 0)),
            pl.BlockSpec((1, NL - 1, HID, HID), lambda i, be, nv: (be[i], 0, 0, 0)),
            pl.BlockSpec((1, NL - 1, HID), lambda i, be, nv: (be[i], 0, 0)),
            pl.BlockSpec((1, NL - 1, HID), lambda i, be, nv: (be[i], 0, 0)),
            pl.BlockSpec((1, HID, PADW), lambda i, be, nv: (be[i], 0, 0)),
            pl.BlockSpec((1, 1, PADW), lambda i, be, nv: (be[i], 0, 0)),
        ],
        out_specs=pl.BlockSpec((B, PADW), lambda i, be, nv: (i, 0)),
    )
    vals = pl.pallas_call(
        _mlp_block,
        grid_spec=grid_spec,
        out_shape=jax.ShapeDtypeStruct((NROWS, PADW), jnp.float32),
    )(block_expert, nvb, tok, f2, w0a, w0sc, b0r, Wb, bb, scl3, wo16, bo16)

    scat = sc_scatter(vals, sidx.reshape(NCHUNK, CHUNK))
    return scat[:N, :OUT]


# speedup vs baseline: 1.0307x; 1.0027x over previous
import functools

import numpy as np
import jax
import jax.numpy as jnp
from jax import lax
from jax.experimental import pallas as pl
from jax.experimental.pallas import tpu as pltpu
from jax.experimental.pallas import tpu_sc as plsc

N = 8192
INPUT_DIM = 4
HID = 256
OUT = 4
NF = 10
NL = 8
NC = 8
B = 256
NBLK = N // B + NC
NROWS = NBLK * B
PADW = 128
CHUNK = 80
NCHUNK = NROWS // CHUNK
NCC, NSC = 2, 16
NW = NCC * NSC
CPW = NCHUNK // NW

_PERM_SIN = np.array([4 + i * 8 + 2 * j for i in range(NF) for j in range(INPUT_DIM)])
_PERM_COS = _PERM_SIN + 1
_F2 = np.zeros((3 * INPUT_DIM, NF * INPUT_DIM), np.float32)
for _i in range(NF):
    for _j in range(INPUT_DIM):
        for _k in range(3):
            _F2[_k * INPUT_DIM + _j, _i * INPUT_DIM + _j] = 2.0 ** _i


def _mlp_block(be_ref, nv_ref, tok_ref, f2_ref, w0a_ref, w0sc_ref, b0_ref,
               wb_ref, bb_ref, sc_ref, wo_ref, bo_ref, out_ref):
    @pl.when(pl.program_id(0) < nv_ref[0])
    def _valid_block():
        _mlp_block_body(tok_ref, f2_ref, w0a_ref, w0sc_ref, b0_ref,
                        wb_ref, bb_ref, sc_ref, wo_ref, bo_ref, out_ref)


def _mlp_block_body(tok_ref, f2_ref, w0a_ref, w0sc_ref, b0_ref,
                    wb_ref, bb_ref, sc_ref, wo_ref, bo_ref, out_ref):
    t = tok_ref[...]
    x4 = t[:, 0:4]
    xn = jnp.where(lax.broadcasted_iota(jnp.int32, (B, INPUT_DIM), 1) < 3,
                   x4 / t[:, 3:4], x4)
    x1 = xn.astype(jnp.bfloat16).astype(jnp.float32)
    r1 = xn - x1
    x2 = r1.astype(jnp.bfloat16).astype(jnp.float32)
    x3 = r1 - x2
    xs = jnp.concatenate([x1, x2, x3], axis=1)
    ang = jnp.dot(xs, f2_ref[...],
                  preferred_element_type=jnp.float32) * np.float32(np.pi)
    angc = jnp.concatenate([ang, ang + np.float32(np.pi / 2)], axis=1)
    s80 = jnp.sin(angc)
    h = (jnp.dot(xn, w0a_ref[0], preferred_element_type=jnp.float32)
         + jnp.dot(s80, w0sc_ref[0], preferred_element_type=jnp.float32)
         + b0_ref[0])
    h = jnp.maximum(h, 0.0)
    ha, hb = h[:B // 2], h[B // 2:]
    for l in range(NL - 1):
        wl, bl, sl = wb_ref[0, l], bb_ref[0, l], sc_ref[0, l]
        za = jnp.dot(ha, wl, preferred_element_type=jnp.float32) + bl
        zb = jnp.dot(hb, wl, preferred_element_type=jnp.float32) + bl
        ha = sl * jnp.maximum(za, 0.0) + ha
        hb = sl * jnp.maximum(zb, 0.0) + hb
    h = jnp.concatenate([ha, hb], axis=0)
    o = jnp.dot(h, wo_ref[0], preferred_element_type=jnp.float32) + bo_ref[0]
    out_ref[...] = o / t[:, 4:5]


@functools.cache
def _sc_kernels():
    mesh = plsc.VectorSubcoreMesh(core_axis_name="c", subcore_axis_name="s")

    @functools.partial(
        pl.kernel, mesh=mesh,
        out_type=jax.ShapeDtypeStruct((NROWS, PADW), jnp.float32),
        scratch_types=[pltpu.VMEM((CPW, CHUNK), jnp.int32),
                       pltpu.VMEM((CPW, CHUNK, PADW), jnp.float32),
                       pltpu.SemaphoreType.DMA,
                       pltpu.SemaphoreType.DMA,
                       pltpu.SemaphoreType.DMA])
    def sc_gather(table, idx2, out, idx_v, rows_v, s1, s2, s3):
        wid = lax.axis_index("s") * NCC + lax.axis_index("c")
        base = wid * CPW
        pltpu.async_copy(idx2.at[pl.ds(base, CPW)], idx_v, s1).wait()
        gcs = [pltpu.async_copy(table.at[idx_v.at[k]], rows_v.at[k], s2)
               for k in range(CPW)]
        wcs = []
        for k in range(CPW):
            gcs[k].wait()
            wcs.append(pltpu.async_copy(
                rows_v.at[k], out.at[pl.ds((base + k) * CHUNK, CHUNK)], s3))
        for w in wcs:
            w.wait()

    @functools.partial(
        pl.kernel, mesh=mesh,
        out_type=jax.ShapeDtypeStruct((N + 8, PADW), jnp.float32),
        scratch_types=[pltpu.VMEM((CPW, CHUNK), jnp.int32),
                       pltpu.VMEM((CPW, CHUNK, PADW), jnp.float32),
                       pltpu.SemaphoreType.DMA,
                       pltpu.SemaphoreType.DMA,
                       pltpu.SemaphoreType.DMA])
    def sc_scatter(vals, idx2, out, idx_v, rows_v, s1, s2, s3):
        wid = lax.axis_index("s") * NCC + lax.axis_index("c")
        base = wid * CPW
        ic = pltpu.async_copy(idx2.at[pl.ds(base, CPW)], idx_v, s1)
        vcs = [pltpu.async_copy(vals.at[pl.ds((base + k) * CHUNK, CHUNK)],
                                rows_v.at[k], s2)
               for k in range(CPW)]
        ic.wait()
        wcs = []
        for k in range(CPW):
            vcs[k].wait()
            wcs.append(pltpu.async_copy(rows_v.at[k], out.at[idx_v.at[k]], s3))
        for w in wcs:
            w.wait()

    return sc_gather, sc_scatter


def _routing(layer_id):
    e = layer_id.astype(jnp.int32)
    onehot = (e[:, None] == jnp.arange(NC, dtype=jnp.int32)).astype(jnp.int32)
    rank = jnp.take_along_axis(jnp.cumsum(onehot, axis=0) - onehot,
                               e[:, None], axis=1)[:, 0]
    counts = jnp.sum(onehot, axis=0)
    padded = ((counts + B - 1) // B) * B
    starts = jnp.concatenate(
        [jnp.zeros((1,), jnp.int32), jnp.cumsum(padded)[:-1].astype(jnp.int32)])
    dest = jnp.take(starts, e) + rank
    sidx = jnp.full((NROWS,), N, jnp.int32).at[dest].set(
        jnp.arange(N, dtype=jnp.int32))
    gidx = jnp.where(sidx == N, 0, sidx)
    block_expert = (jnp.searchsorted(
        starts, jnp.arange(NBLK, dtype=jnp.int32) * B, side="right") - 1
    ).astype(jnp.int32)
    nvb = (jnp.sum(padded) // B).astype(jnp.int32)[None]
    return gidx, sidx, block_expert, nvb


def kernel(x, layer_id, input_dim, W0, b0, Wb, bb, scalars, Wo, bo):
    gidx, sidx, block_expert, nvb = _routing(layer_id)

    xpad = jnp.concatenate(
        [x, input_dim[:, None], jnp.zeros((N, PADW - INPUT_DIM - 1), jnp.float32)],
        axis=1)

    sc_gather, sc_scatter = _sc_kernels()
    tok = sc_gather(xpad, gidx.reshape(NCHUNK, CHUNK))

    f2 = jnp.asarray(_F2)
    w0a = W0[:, :INPUT_DIM, :]
    w0sc = W0[:, np.concatenate([_PERM_SIN, _PERM_COS]), :]
    b0r = b0[:, None, :]
    scl3 = jnp.broadcast_to(scalars[:, :, None], (NC, NL - 1, HID))
    wo16 = jnp.zeros((NC, HID, PADW), jnp.float32).at[:, :, :OUT].set(Wo)
    bo16 = jnp.zeros((NC, 1, PADW), jnp.float32).at[:, 0, :OUT].set(bo)

    grid_spec = pltpu.PrefetchScalarGridSpec(
        num_scalar_prefetch=2,
        grid=(NBLK,),
        in_specs=[
            pl.BlockSpec((B, PADW), lambda i, be, nv: (i, 0)),
            pl.BlockSpec((3 * INPUT_DIM, NF * INPUT_DIM), lambda i, be, nv: (0, 0)),
            pl.BlockSpec((1, INPUT_DIM, HID), lambda i, be, nv: (be[i], 0, 0)),
            pl.BlockSpec((1, 2 * NF * INPUT_DIM, HID), lambda i, be, nv: (be[i], 0, 0)),
            pl.BlockSpec((1, 1, HID), lambda i, be, nv: (be[i], 0, 0)),
            pl.BlockSpec((1, NL - 1, HID, HID), lambda i, be, nv: (be[i], 0, 0, 0)),
            pl.BlockSpec((1, NL - 1, HID), lambda i, be, nv: (be[i], 0, 0)),
            pl.BlockSpec((1, NL - 1, HID), lambda i, be, nv: (be[i], 0, 0)),
            pl.BlockSpec((1, HID, PADW), lambda i, be, nv: (be[i], 0, 0)),
            pl.BlockSpec((1, 1, PADW), lambda i, be, nv: (be[i], 0, 0)),
        ],
        out_specs=pl.BlockSpec((B, PADW), lambda i, be, nv: (i, 0)),
    )
    vals = pl.pallas_call(
        _mlp_block,
        grid_spec=grid_spec,
        out_shape=jax.ShapeDtypeStruct((NROWS, PADW), jnp.float32),
    )(block_expert, nvb, tok, f2, w0a, w0sc, b0r, Wb, bb, scl3, wo16, bo16)

    scat = sc_scatter(vals, sidx.reshape(NCHUNK, CHUNK))
    return scat[:N, :OUT]


# speedup vs baseline: 1.0561x; 1.0246x over previous
import functools

import numpy as np
import jax
import jax.numpy as jnp
from jax import lax
from jax.experimental import pallas as pl
from jax.experimental.pallas import tpu as pltpu
from jax.experimental.pallas import tpu_sc as plsc

N = 8192
INPUT_DIM = 4
HID = 256
OUT = 4
NF = 10
NL = 8
NC = 8
B = 256
NBLK = N // B + NC
NROWS = NBLK * B
PADW = 128
CHUNK = 80
NCHUNK = NROWS // CHUNK
NCC, NSC = 2, 16
NW = NCC * NSC
CPW = NCHUNK // NW

_PERM_SIN = np.array([4 + i * 8 + 2 * j for i in range(NF) for j in range(INPUT_DIM)])
_PERM_COS = _PERM_SIN + 1
_F2 = np.zeros((3 * INPUT_DIM, NF * INPUT_DIM), np.float32)
for _i in range(NF):
    for _j in range(INPUT_DIM):
        for _k in range(3):
            _F2[_k * INPUT_DIM + _j, _i * INPUT_DIM + _j] = 2.0 ** _i


def _mlp_block(be_ref, nv_ref, tok_ref, f2_ref, w0a_ref, w0sc_ref, b0_ref,
               wb_ref, bb_ref, sc_ref, wo_ref, bo_ref, out_ref):
    @pl.when(pl.program_id(0) < nv_ref[0])
    def _valid_block():
        _mlp_block_body(tok_ref, f2_ref, w0a_ref, w0sc_ref, b0_ref,
                        wb_ref, bb_ref, sc_ref, wo_ref, bo_ref, out_ref)


def _mlp_block_body(tok_ref, f2_ref, w0a_ref, w0sc_ref, b0_ref,
                    wb_ref, bb_ref, sc_ref, wo_ref, bo_ref, out_ref):
    t = tok_ref[...]
    x4 = t[:, 0:4]
    xn = jnp.where(lax.broadcasted_iota(jnp.int32, (B, INPUT_DIM), 1) < 3,
                   x4 / t[:, 3:4], x4)
    x1 = xn.astype(jnp.bfloat16).astype(jnp.float32)
    r1 = xn - x1
    x2 = r1.astype(jnp.bfloat16).astype(jnp.float32)
    x3 = r1 - x2
    xs = jnp.concatenate([x1, x2, x3], axis=1)
    ang = jnp.dot(xs, f2_ref[...],
                  preferred_element_type=jnp.float32) * np.float32(np.pi)
    angc = jnp.concatenate([ang, ang + np.float32(np.pi / 2)], axis=1)
    s80 = jnp.sin(angc)
    h = (jnp.dot(xn, w0a_ref[0], preferred_element_type=jnp.float32)
         + jnp.dot(s80, w0sc_ref[0], preferred_element_type=jnp.float32)
         + b0_ref[0])
    h = jnp.maximum(h, 0.0)
    ha, hb = h[:B // 2], h[B // 2:]
    for l in range(NL - 1):
        wl, bl, sl = wb_ref[0, l], bb_ref[0, l], sc_ref[0, l]
        za = jnp.dot(ha, wl, preferred_element_type=jnp.float32) + bl
        zb = jnp.dot(hb, wl, preferred_element_type=jnp.float32) + bl
        ha = sl * jnp.maximum(za, 0.0) + ha
        hb = sl * jnp.maximum(zb, 0.0) + hb
    h = jnp.concatenate([ha, hb], axis=0)
    o = jnp.dot(h, wo_ref[0], preferred_element_type=jnp.float32) + bo_ref[0]
    out_ref[...] = o / t[:, 4:5]


@functools.cache
def _sc_kernels():
    mesh = plsc.VectorSubcoreMesh(core_axis_name="c", subcore_axis_name="s")

    @functools.partial(
        pl.kernel, mesh=mesh,
        out_type=jax.ShapeDtypeStruct((NROWS, PADW), jnp.float32),
        scratch_types=[pltpu.VMEM((CPW, CHUNK), jnp.int32),
                       pltpu.VMEM((CPW, CHUNK, PADW), jnp.float32),
                       pltpu.SemaphoreType.DMA,
                       pltpu.SemaphoreType.DMA,
                       pltpu.SemaphoreType.DMA])
    def sc_gather(table, idx2, out, idx_v, rows_v, s1, s2, s3):
        wid = lax.axis_index("s") * NCC + lax.axis_index("c")
        base = wid * CPW
        pltpu.async_copy(idx2.at[pl.ds(base, CPW)], idx_v, s1).wait()
        gcs = [pltpu.async_copy(table.at[idx_v.at[k]], rows_v.at[k], s2)
               for k in range(CPW)]
        wcs = []
        for k in range(CPW):
            gcs[k].wait()
            wcs.append(pltpu.async_copy(
                rows_v.at[k], out.at[pl.ds((base + k) * CHUNK, CHUNK)], s3))
        for w in wcs:
            w.wait()

    @functools.partial(
        pl.kernel, mesh=mesh,
        out_type=jax.ShapeDtypeStruct((N + 8, PADW), jnp.float32),
        scratch_types=[pltpu.VMEM((CPW, CHUNK), jnp.int32),
                       pltpu.VMEM((CPW, CHUNK, PADW), jnp.float32),
                       pltpu.SemaphoreType.DMA,
                       pltpu.SemaphoreType.DMA,
                       pltpu.SemaphoreType.DMA])
    def sc_scatter(vals, idx2, out, idx_v, rows_v, s1, s2, s3):
        wid = lax.axis_index("s") * NCC + lax.axis_index("c")
        base = wid * CPW
        ic = pltpu.async_copy(idx2.at[pl.ds(base, CPW)], idx_v, s1)
        vcs = [pltpu.async_copy(vals.at[pl.ds((base + k) * CHUNK, CHUNK)],
                                rows_v.at[k], s2)
               for k in range(CPW)]
        ic.wait()
        wcs = []
        for k in range(CPW):
            vcs[k].wait()
            wcs.append(pltpu.async_copy(rows_v.at[k], out.at[idx_v.at[k]], s3))
        for w in wcs:
            w.wait()

    return sc_gather, sc_scatter


def _routing(layer_id):
    e = layer_id.astype(jnp.int32)
    ohb = e[:, None] == jnp.arange(NC, dtype=jnp.int32)
    onehot = ohb.astype(jnp.int32)
    rank = jnp.sum((jnp.cumsum(onehot, axis=0) - onehot) * onehot, axis=1)
    counts = jnp.sum(onehot, axis=0)
    padded = ((counts + B - 1) // B) * B
    starts = jnp.concatenate(
        [jnp.zeros((1,), jnp.int32), jnp.cumsum(padded)[:-1].astype(jnp.int32)])
    dest = jnp.sum(jnp.where(ohb, starts[None, :], 0), axis=1) + rank
    sidx = jnp.full((NROWS,), N, jnp.int32).at[dest].set(
        jnp.arange(N, dtype=jnp.int32))
    gidx = jnp.where(sidx == N, 0, sidx)
    block_expert = (jnp.searchsorted(
        starts, jnp.arange(NBLK, dtype=jnp.int32) * B, side="right") - 1
    ).astype(jnp.int32)
    nvb = (jnp.sum(padded) // B).astype(jnp.int32)[None]
    return gidx, sidx, block_expert, nvb


def kernel(x, layer_id, input_dim, W0, b0, Wb, bb, scalars, Wo, bo):
    gidx, sidx, block_expert, nvb = _routing(layer_id)

    xpad = jnp.concatenate(
        [x, input_dim[:, None], jnp.zeros((N, PADW - INPUT_DIM - 1), jnp.float32)],
        axis=1)

    sc_gather, sc_scatter = _sc_kernels()
    tok = sc_gather(xpad, gidx.reshape(NCHUNK, CHUNK))

    f2 = jnp.asarray(_F2)
    w0a = W0[:, :INPUT_DIM, :]
    w0sc = W0[:, np.concatenate([_PERM_SIN, _PERM_COS]), :]
    b0r = b0[:, None, :]
    scl3 = jnp.broadcast_to(scalars[:, :, None], (NC, NL - 1, HID))
    wo16 = jnp.zeros((NC, HID, PADW), jnp.float32).at[:, :, :OUT].set(Wo)
    bo16 = jnp.zeros((NC, 1, PADW), jnp.float32).at[:, 0, :OUT].set(bo)

    grid_spec = pltpu.PrefetchScalarGridSpec(
        num_scalar_prefetch=2,
        grid=(NBLK,),
        in_specs=[
            pl.BlockSpec((B, PADW), lambda i, be, nv: (i, 0)),
            pl.BlockSpec((3 * INPUT_DIM, NF * INPUT_DIM), lambda i, be, nv: (0, 0)),
            pl.BlockSpec((1, INPUT_DIM, HID), lambda i, be, nv: (be[i], 0, 0)),
            pl.BlockSpec((1, 2 * NF * INPUT_DIM, HID), lambda i, be, nv: (be[i], 0, 0)),
            pl.BlockSpec((1, 1, HID), lambda i, be, nv: (be[i], 0, 0)),
            pl.BlockSpec((1, NL - 1, HID, HID), lambda i, be, nv: (be[i], 0, 0, 0)),
            pl.BlockSpec((1, NL - 1, HID), lambda i, be, nv: (be[i], 0, 0)),
            pl.BlockSpec((1, NL - 1, HID), lambda i, be, nv: (be[i], 0, 0)),
            pl.BlockSpec((1, HID, PADW), lambda i, be, nv: (be[i], 0, 0)),
            pl.BlockSpec((1, 1, PADW), lambda i, be, nv: (be[i], 0, 0)),
        ],
        out_specs=pl.BlockSpec((B, PADW), lambda i, be, nv: (i, 0)),
    )
    vals = pl.pallas_call(
        _mlp_block,
        grid_spec=grid_spec,
        out_shape=jax.ShapeDtypeStruct((NROWS, PADW), jnp.float32),
    )(block_expert, nvb, tok, f2, w0a, w0sc, b0r, Wb, bb, scl3, wo16, bo16)

    scat = sc_scatter(vals, sidx.reshape(NCHUNK, CHUNK))
    return scat[:N, :OUT]
